# baseline (speedup 1.0000x reference)
import functools

import jax
import jax.numpy as jnp
from jax import lax
from jax.experimental import pallas as pl
from jax.experimental.pallas import tpu as pltpu
from jax.experimental.pallas import tpu_sc as plsc

N = 10000
E = 160000
D = 256
DE = 16
H = 512

NC = 2
NS = 16
NW = NC * NS

_MESH = lambda: plsc.VectorSubcoreMesh(
    core_axis_name="c", subcore_axis_name="s", num_cores=NC, num_subcores=NS)

EPW = E // NW
GB = 200


DP = D // 2


RING = 4
K = 5
EC = E // K
EPWC = EC // NW
NCH = EPWC // GB


def _sc_gather(nodes_u, senders, receivers, k):
  @functools.partial(
      pl.kernel,
      out_type=(jax.ShapeDtypeStruct((EC, DP), jnp.uint32),
                jax.ShapeDtypeStruct((EC, DP), jnp.uint32)),
      mesh=_MESH(),
      scratch_types=[
          pltpu.VMEM((EPWC,), jnp.int32),
          pltpu.VMEM((EPWC,), jnp.int32),
          pltpu.VMEM((RING, GB, DP), jnp.uint32),
          pltpu.SemaphoreType.DMA((RING,)),
          pltpu.SemaphoreType.DMA((RING,)),
      ],
  )
  def kern(nodes_hbm, snd_hbm, rcv_hbm, gs_hbm, gr_hbm, idx_s, idx_r, buf,
           gsem, wsem):
    wid = lax.axis_index("s") * NC + lax.axis_index("c")
    base = wid * EPWC
    pltpu.sync_copy(snd_hbm.at[pl.ds(k * EC + base, EPWC)], idx_s)
    pltpu.sync_copy(rcv_hbm.at[pl.ds(k * EC + base, EPWC)], idx_r)

    def run(idx_v, out_hbm):
      def g_desc(i):
        par = lax.rem(i, RING)
        return pltpu.make_async_copy(
            nodes_hbm.at[idx_v.at[pl.ds(i * GB, GB)]], buf.at[par],
            gsem.at[par])

      def w_desc(i):
        par = lax.rem(i, RING)
        return pltpu.make_async_copy(
            buf.at[par], out_hbm.at[pl.ds(base + i * GB, GB)], wsem.at[par])

      for j in range(RING):
        g_desc(j).start()

      def body(i, carry):
        g_desc(i).wait()
        w_desc(i).start()

        @pl.when(jnp.logical_and(i >= 1, i + RING - 1 < NCH))
        def _():
          w_desc(i - 1).wait()
          g_desc(i + RING - 1).start()

        return carry

      lax.fori_loop(0, NCH, body, 0)
      for j in range(RING):
        w_desc(NCH - RING + j).wait()

    run(idx_s, gs_hbm)
    run(idx_r, gr_hbm)

  return kern(nodes_u, senders, receivers)


EPT = EC // NS
SB = 80
NCHS = EPT // SB
NPAD = 10240
NPT = NPAD // NS
DH = D // NC
NLAST = N - (NS - 1) * NPT


def _sc_scatter(ue_k, receivers, init, k):
  @functools.partial(
      pl.kernel,
      out_type=jax.ShapeDtypeStruct((N, D), jnp.float32),
      mesh=_MESH(),
      scratch_types=[
          pltpu.VMEM((RING, SB), jnp.int32),
          pltpu.VMEM((RING, SB, DH), jnp.float32),
          pltpu.VMEM_SHARED((NPAD, DH), jnp.float32),
          pltpu.SemaphoreType.DMA((RING,)),
          pltpu.SemaphoreType.DMA((RING,)),
      ],
  )
  def kern(ue_hbm, rcv_hbm, init_hbm, agg_hbm, idx_v, rows_v, acc_sh, lsem,
           asem):
    c = lax.axis_index("c")
    s = lax.axis_index("s")
    col = pl.multiple_of(c * DH, DH)
    @pl.when(s < NS - 1)
    def _():
      pltpu.sync_copy(init_hbm.at[pl.ds(s * NPT, NPT), pl.ds(col, DH)],
                      acc_sh.at[pl.ds(s * NPT, NPT)])

    @pl.when(s == NS - 1)
    def _():
      pltpu.sync_copy(init_hbm.at[pl.ds((NS - 1) * NPT, NLAST),
                                  pl.ds(col, DH)],
                      acc_sh.at[pl.ds((NS - 1) * NPT, NLAST)])

    plsc.subcore_barrier()
    ebase = s * EPT

    def idx_desc(i):
      par = lax.rem(i, RING)
      off = k * EC + ebase + i * SB
      return pltpu.make_async_copy(rcv_hbm.at[pl.ds(off, SB)],
                                   idx_v.at[par], lsem.at[par])

    def rows_desc(i):
      par = lax.rem(i, RING)
      off = ebase + i * SB
      return pltpu.make_async_copy(
          ue_hbm.at[pl.ds(off, SB), pl.ds(col, DH)], rows_v.at[par],
          lsem.at[par])

    def add_start(i):
      par = lax.rem(i, RING)
      pltpu.async_copy(rows_v.at[par], acc_sh.at[idx_v.at[par]],
                       asem.at[par], add=True)

    def add_wait(i):
      par = lax.rem(i, RING)
      pltpu.make_async_copy(rows_v.at[par], acc_sh.at[idx_v.at[par]],
                            asem.at[par]).wait()

    def fire_load(i):
      idx_desc(i).start()
      rows_desc(i).start()

    for j in range(RING):
      fire_load(j)

    def body(i, carry):
      idx_desc(i).wait()
      rows_desc(i).wait()
      add_start(i)

      @pl.when(jnp.logical_and(i >= 1, i + RING - 1 < NCHS))
      def _():
        add_wait(i - 1)
        fire_load(i + RING - 1)

      return carry

    lax.fori_loop(0, NCHS, body, 0)
    for j in range(RING):
      add_wait(NCHS - RING + j)
    plsc.subcore_barrier()
    @pl.when(s < NS - 1)
    def _():
      pltpu.sync_copy(acc_sh.at[pl.ds(s * NPT, NPT)],
                      agg_hbm.at[pl.ds(s * NPT, NPT), pl.ds(col, DH)])

    @pl.when(s == NS - 1)
    def _():
      pltpu.sync_copy(acc_sh.at[pl.ds((NS - 1) * NPT, NLAST)],
                      agg_hbm.at[pl.ds((NS - 1) * NPT, NLAST),
                                 pl.ds(col, DH)])

  return kern(ue_k, receivers, init)


BP = 1000


def _tc_pack(nodes):
  def body(n_r, out_r):
    xb = jax.lax.bitcast_convert_type(n_r[...], jnp.uint32)
    xb = xb + jnp.uint32(0x8000)
    out_r[...] = (xb[:, :DP] >> 16) | (xb[:, DP:] & jnp.uint32(0xFFFF0000))

  return pl.pallas_call(
      body,
      grid=(N // BP,),
      in_specs=[pl.BlockSpec((BP, D), lambda i: (i, 0))],
      out_specs=pl.BlockSpec((BP, DP), lambda i: (i, 0)),
      out_shape=jax.ShapeDtypeStruct((N, DP), jnp.uint32),
  )(nodes)


BE = 800


def _tc_edge_mlp(gs, gr, ed, w1, b1, w2, b2, k):
  def body(gs_r, gr_r, ed_r, w1_r, b1_r, w2_r, b2_r, out_r):
    bf = jnp.bfloat16
    hi = jnp.uint32(0xFFFF0000)

    def unpack(u):
      lo = jax.lax.bitcast_convert_type(u << 16, jnp.float32).astype(bf)
      up = jax.lax.bitcast_convert_type(u & hi, jnp.float32).astype(bf)
      return lo, up

    gsl, gsu = unpack(gs_r[...])
    grl, gru = unpack(gr_r[...])
    x = jnp.concatenate([gsl, gsu, grl, gru, ed_r[...]], axis=1)
    h = jnp.dot(x, w1_r[...], preferred_element_type=jnp.float32)
    h = jnp.maximum(h + b1_r[...], 0.0).astype(bf)
    out_r[...] = (jnp.dot(h, w2_r[...], preferred_element_type=jnp.float32)
                  + b2_r[...])

  full = lambda shape: pl.BlockSpec(shape, lambda i: (0, 0))
  koff = k * (EC // BE)
  return pl.pallas_call(
      body,
      grid=(EC // BE,),
      in_specs=[
          pl.BlockSpec((BE, DP), lambda i: (i, 0)),
          pl.BlockSpec((BE, DP), lambda i: (i, 0)),
          pl.BlockSpec((BE, DE), lambda i: (koff + i, 0)),
          full((2 * D + DE, H)),
          full((1, H)),
          full((H, D)),
          full((1, D)),
      ],
      out_specs=pl.BlockSpec((BE, D), lambda i: (i, 0)),
      out_shape=jax.ShapeDtypeStruct((EC, D), jnp.float32),
  )(gs, gr, ed, w1, b1, w2, b2)


BN = 1000


def _tc_node_mlp(nodes, agg, w1a, w1b, b1, w2, b2):
  def body(n_r, a_r, w1a_r, w1b_r, b1_r, w2_r, b2_r, out_r):
    h = jnp.dot(n_r[...], w1a_r[...], preferred_element_type=jnp.float32)
    h = h + jnp.dot(a_r[...].astype(jnp.bfloat16), w1b_r[...],
                    preferred_element_type=jnp.float32)
    h = jnp.maximum(h + b1_r[...], 0.0).astype(jnp.bfloat16)
    out_r[...] = (jnp.dot(h, w2_r[...], preferred_element_type=jnp.float32)
                  + b2_r[...])

  full = lambda shape: pl.BlockSpec(shape, lambda i: (0, 0))
  return pl.pallas_call(
      body,
      grid=(N // BN,),
      in_specs=[
          pl.BlockSpec((BN, D), lambda i: (i, 0)),
          pl.BlockSpec((BN, D), lambda i: (i, 0)),
          full((D, H)),
          full((D, H)),
          full((1, H)),
          full((H, D)),
          full((1, D)),
      ],
      out_specs=pl.BlockSpec((BN, D), lambda i: (i, 0)),
      out_shape=jax.ShapeDtypeStruct((N, D), jnp.float32),
  )(nodes, agg, w1a, w1b, b1, w2, b2)


def kernel(nodes, edges, senders, receivers,
           We1, be1, We2, be2, Wn1, bn1, Wn2, bn2):
  bf = jnp.bfloat16
  We1b, We2b = We1.astype(bf), We2.astype(bf)
  edges_b = edges.astype(bf)
  nodes_u = _tc_pack(nodes)
  gath = [_sc_gather(nodes_u, senders, receivers, k) for k in range(K)]
  ues = [_tc_edge_mlp(gath[k][0], gath[k][1], edges_b, We1b,
                      be1.reshape(1, H), We2b, be2.reshape(1, D), k)
         for k in range(K)]
  agg = jnp.zeros((N, D), jnp.float32)
  for k in range(K):
    agg = _sc_scatter(ues[k], receivers, agg, k)
  ue = jnp.concatenate(ues, axis=0)
  Wn1b, Wn2b = Wn1.astype(bf), Wn2.astype(bf)
  un = _tc_node_mlp(nodes.astype(bf), agg, Wn1b[:D], Wn1b[D:],
                    bn1.reshape(1, H), Wn2b, bn2.reshape(1, D))
  return (un, ue)


# speedup vs baseline: 1.0026x; 1.0026x over previous
import functools

import jax
import jax.numpy as jnp
from jax import lax
from jax.experimental import pallas as pl
from jax.experimental.pallas import tpu as pltpu
from jax.experimental.pallas import tpu_sc as plsc

N = 10000
E = 160000
D = 256
DE = 16
H = 512

NC = 2
NS = 16
NW = NC * NS

_MESH = lambda: plsc.VectorSubcoreMesh(
    core_axis_name="c", subcore_axis_name="s", num_cores=NC, num_subcores=NS)

EPW = E // NW
GB = 200


DP = D // 2


RING = 4
NCH = EPW // GB


def _sc_gather(nodes_u, senders, receivers):
  @functools.partial(
      pl.kernel,
      out_type=(jax.ShapeDtypeStruct((E, DP), jnp.uint32),
                jax.ShapeDtypeStruct((E, DP), jnp.uint32)),
      mesh=_MESH(),
      scratch_types=[
          pltpu.VMEM((EPW,), jnp.int32),
          pltpu.VMEM((EPW,), jnp.int32),
          pltpu.VMEM((RING, GB, DP), jnp.uint32),
          pltpu.SemaphoreType.DMA((RING,)),
          pltpu.SemaphoreType.DMA((RING,)),
      ],
  )
  def k(nodes_hbm, snd_hbm, rcv_hbm, gs_hbm, gr_hbm, idx_s, idx_r, buf,
        gsem, wsem):
    wid = lax.axis_index("s") * NC + lax.axis_index("c")
    base = wid * EPW
    pltpu.sync_copy(snd_hbm.at[pl.ds(base, EPW)], idx_s)
    pltpu.sync_copy(rcv_hbm.at[pl.ds(base, EPW)], idx_r)

    def run(idx_v, out_hbm):
      def g_desc(i):
        par = lax.rem(i, RING)
        return pltpu.make_async_copy(
            nodes_hbm.at[idx_v.at[pl.ds(i * GB, GB)]], buf.at[par],
            gsem.at[par])

      def w_desc(i):
        par = lax.rem(i, RING)
        return pltpu.make_async_copy(
            buf.at[par], out_hbm.at[pl.ds(base + i * GB, GB)], wsem.at[par])

      for j in range(RING):
        g_desc(j).start()

      def body(i, carry):
        g_desc(i).wait()
        w_desc(i).start()

        @pl.when(jnp.logical_and(i >= 1, i + RING - 1 < NCH))
        def _():
          w_desc(i - 1).wait()
          g_desc(i + RING - 1).start()

        return carry

      lax.fori_loop(0, NCH, body, 0)
      for j in range(RING):
        w_desc(NCH - RING + j).wait()

    run(idx_s, gs_hbm)
    run(idx_r, gr_hbm)

  return k(nodes_u, senders, receivers)


EPT = E // NS
SB = 80
NCHS = EPT // SB
NPAD = 10240
NPT = NPAD // NS
DH = D // NC
NLAST = N - (NS - 1) * NPT


def _sc_scatter(ue, receivers, zeros_half):
  @functools.partial(
      pl.kernel,
      out_type=jax.ShapeDtypeStruct((N, D), jnp.float32),
      mesh=_MESH(),
      scratch_types=[
          pltpu.VMEM((RING, SB), jnp.int32),
          pltpu.VMEM((RING, SB, DH), jnp.float32),
          pltpu.VMEM_SHARED((NPAD, DH), jnp.float32),
          pltpu.SemaphoreType.DMA((RING,)),
          pltpu.SemaphoreType.DMA((RING,)),
      ],
  )
  def k(ue_hbm, rcv_hbm, zero_hbm, agg_hbm, idx_v, rows_v, acc_sh, lsem,
        asem):
    c = lax.axis_index("c")
    s = lax.axis_index("s")
    col = pl.multiple_of(c * DH, DH)
    pltpu.sync_copy(zero_hbm, acc_sh.at[pl.ds(s * NPT, NPT)])
    plsc.subcore_barrier()
    ebase = s * EPT

    def idx_desc(i):
      par = lax.rem(i, RING)
      off = ebase + i * SB
      return pltpu.make_async_copy(rcv_hbm.at[pl.ds(off, SB)],
                                   idx_v.at[par], lsem.at[par])

    def rows_desc(i):
      par = lax.rem(i, RING)
      off = ebase + i * SB
      return pltpu.make_async_copy(
          ue_hbm.at[pl.ds(off, SB), pl.ds(col, DH)], rows_v.at[par],
          lsem.at[par])

    def add_start(i):
      par = lax.rem(i, RING)
      pltpu.async_copy(rows_v.at[par], acc_sh.at[idx_v.at[par]],
                       asem.at[par], add=True)

    def add_wait(i):
      par = lax.rem(i, RING)
      pltpu.make_async_copy(rows_v.at[par], acc_sh.at[idx_v.at[par]],
                            asem.at[par]).wait()

    def fire_load(i):
      idx_desc(i).start()
      rows_desc(i).start()

    for j in range(RING):
      fire_load(j)

    def body(i, carry):
      idx_desc(i).wait()
      rows_desc(i).wait()
      add_start(i)

      @pl.when(jnp.logical_and(i >= 1, i + RING - 1 < NCHS))
      def _():
        add_wait(i - 1)
        fire_load(i + RING - 1)

      return carry

    lax.fori_loop(0, NCHS, body, 0)
    for j in range(RING):
      add_wait(NCHS - RING + j)
    plsc.subcore_barrier()
    @pl.when(s < NS - 1)
    def _():
      pltpu.sync_copy(acc_sh.at[pl.ds(s * NPT, NPT)],
                      agg_hbm.at[pl.ds(s * NPT, NPT), pl.ds(col, DH)])

    @pl.when(s == NS - 1)
    def _():
      pltpu.sync_copy(acc_sh.at[pl.ds((NS - 1) * NPT, NLAST)],
                      agg_hbm.at[pl.ds((NS - 1) * NPT, NLAST),
                                 pl.ds(col, DH)])

  return k(ue, receivers, zeros_half)


BP = 1000


def _tc_pack(nodes):
  def body(n_r, out_r):
    xb = jax.lax.bitcast_convert_type(n_r[...], jnp.uint32)
    xb = xb + jnp.uint32(0x8000)
    out_r[...] = (xb[:, :DP] >> 16) | (xb[:, DP:] & jnp.uint32(0xFFFF0000))

  return pl.pallas_call(
      body,
      grid=(N // BP,),
      in_specs=[pl.BlockSpec((BP, D), lambda i: (i, 0))],
      out_specs=pl.BlockSpec((BP, DP), lambda i: (i, 0)),
      out_shape=jax.ShapeDtypeStruct((N, DP), jnp.uint32),
  )(nodes)


BE = 800


def _tc_edge_mlp(gs, gr, ed, w1, b1, w2, b2):
  def body(gs_r, gr_r, ed_r, w1_r, b1_r, w2_r, b2_r, out_r):
    bf = jnp.bfloat16
    hi = jnp.uint32(0xFFFF0000)

    def unpack(u):
      lo = jax.lax.bitcast_convert_type(u << 16, jnp.float32).astype(bf)
      up = jax.lax.bitcast_convert_type(u & hi, jnp.float32).astype(bf)
      return lo, up

    gsl, gsu = unpack(gs_r[...])
    grl, gru = unpack(gr_r[...])
    x = jnp.concatenate([gsl, gsu, grl, gru, ed_r[...]], axis=1)
    h = jnp.dot(x, w1_r[...], preferred_element_type=jnp.float32)
    h = jnp.maximum(h + b1_r[...], 0.0).astype(bf)
    out_r[...] = (jnp.dot(h, w2_r[...], preferred_element_type=jnp.float32)
                  + b2_r[...])

  full = lambda shape: pl.BlockSpec(shape, lambda i: (0, 0))
  return pl.pallas_call(
      body,
      grid=(E // BE,),
      in_specs=[
          pl.BlockSpec((BE, DP), lambda i: (i, 0)),
          pl.BlockSpec((BE, DP), lambda i: (i, 0)),
          pl.BlockSpec((BE, DE), lambda i: (i, 0)),
          full((2 * D + DE, H)),
          full((1, H)),
          full((H, D)),
          full((1, D)),
      ],
      out_specs=pl.BlockSpec((BE, D), lambda i: (i, 0)),
      out_shape=jax.ShapeDtypeStruct((E, D), jnp.float32),
  )(gs, gr, ed, w1, b1, w2, b2)


BN = 1000


def _tc_node_mlp(nodes, agg, w1a, w1b, b1, w2, b2):
  def body(n_r, a_r, w1a_r, w1b_r, b1_r, w2_r, b2_r, out_r):
    h = jnp.dot(n_r[...], w1a_r[...], preferred_element_type=jnp.float32)
    h = h + jnp.dot(a_r[...].astype(jnp.bfloat16), w1b_r[...],
                    preferred_element_type=jnp.float32)
    h = jnp.maximum(h + b1_r[...], 0.0).astype(jnp.bfloat16)
    out_r[...] = (jnp.dot(h, w2_r[...], preferred_element_type=jnp.float32)
                  + b2_r[...])

  full = lambda shape: pl.BlockSpec(shape, lambda i: (0, 0))
  return pl.pallas_call(
      body,
      grid=(N // BN,),
      in_specs=[
          pl.BlockSpec((BN, D), lambda i: (i, 0)),
          pl.BlockSpec((BN, D), lambda i: (i, 0)),
          full((D, H)),
          full((D, H)),
          full((1, H)),
          full((H, D)),
          full((1, D)),
      ],
      out_specs=pl.BlockSpec((BN, D), lambda i: (i, 0)),
      out_shape=jax.ShapeDtypeStruct((N, D), jnp.float32),
  )(nodes, agg, w1a, w1b, b1, w2, b2)


def kernel(nodes, edges, senders, receivers,
           We1, be1, We2, be2, Wn1, bn1, Wn2, bn2):
  bf = jnp.bfloat16
  We1b, We2b = We1.astype(bf), We2.astype(bf)
  nodes_u = _tc_pack(nodes)
  gs, gr = _sc_gather(nodes_u, senders, receivers)
  ue = _tc_edge_mlp(gs, gr, edges.astype(bf), We1b,
                    be1.reshape(1, H), We2b, be2.reshape(1, D))
  zeros_half = jnp.zeros((NPT, DH), jnp.float32)
  agg = _sc_scatter(ue, receivers, zeros_half)
  Wn1b, Wn2b = Wn1.astype(bf), Wn2.astype(bf)
  un = _tc_node_mlp(nodes.astype(bf), agg, Wn1b[:D], Wn1b[D:],
                    bn1.reshape(1, H), Wn2b, bn2.reshape(1, D))
  return (un, ue)
